# baseline (speedup 1.0000x reference)
import jax
import jax.numpy as jnp
from jax import lax
from jax.experimental import pallas as pl
from jax.experimental.pallas import tpu as pltpu
from jax.experimental.pallas import tpu_sc as plsc

_NC = 2
_NS = 16
_NW = _NC * _NS

_CHUNK = 256
_IPG = 128
_GPC = _CHUNK // _IPG
_RING = 4

_TPACK = 250


def _pack_body(x_ref, y_ref):
    x = x_ref[...]
    for i in range(8):
        a = x[:, (2 * i) // 8, (2 * i) % 8, :]
        b = x[:, (2 * i + 1) // 8, (2 * i + 1) % 8, :]
        y_ref[:, i, :] = jnp.concatenate([a, b], axis=1)


def _pack(w4):
    G = w4.shape[0]
    return pl.pallas_call(
        _pack_body,
        grid=(G // _TPACK,),
        in_specs=[pl.BlockSpec((_TPACK, 2, 8, 64), lambda j: (j, 0, 0, 0))],
        out_specs=pl.BlockSpec((_TPACK, 8, 128), lambda j: (j, 0, 0)),
        out_shape=jax.ShapeDtypeStruct((G, 8, 128), jnp.float32),
    )(w4)


def _body(idx_hbm, table_hbm, out_hbm, idx_v, rows_v, gsem, osem):
    cpw = idx_hbm.shape[1] // _GPC
    wid = lax.axis_index("s") * _NC + lax.axis_index("c")
    pltpu.sync_copy(idx_hbm.at[wid], idx_v)
    out_base = wid * cpw

    def fire_gathers(g, b):
        for j in range(_GPC):
            pltpu.async_copy(
                table_hbm.at[idx_v.at[g * _GPC + j]],
                rows_v.at[b, pl.ds(j * _IPG, _IPG)],
                gsem.at[b],
            )

    def drain_gathers(b):
        pltpu.make_async_copy(
            out_hbm.at[pl.ds(0, _CHUNK)], rows_v.at[b], gsem.at[b]
        ).wait()

    def out_slice(g):
        base = pl.multiple_of((out_base + g) * _CHUNK, _CHUNK)
        return out_hbm.at[pl.ds(base, _CHUNK)]

    for b in range(_RING):
        fire_gathers(b, b)

    @pl.loop(0, cpw - _RING, step=_RING)
    def _steady(i):
        for b in range(_RING):
            g = i + b
            drain_gathers(b)
            pltpu.async_copy(rows_v.at[b], out_slice(g), osem.at[b])
            pltpu.make_async_copy(
                rows_v.at[b], out_hbm.at[pl.ds(0, _CHUNK)], osem.at[b]
            ).wait()
            fire_gathers(g + _RING, b)

    for b in range(_RING):
        g = cpw - _RING + b
        drain_gathers(b)
        pltpu.sync_copy(rows_v.at[b], out_slice(g))


def kernel(token_ids, W):
    S, T = token_ids.shape
    D = W.shape[1]
    B = S * T
    rows_per_w = B // _NW
    idx = token_ids.astype(jnp.int32).reshape(_NW, rows_per_w // _IPG, _IPG)

    w4 = W.reshape(W.shape[0] // 16, 2, 8, D)
    w_rows = _pack(w4).reshape(W.shape[0], D)

    mesh = plsc.VectorSubcoreMesh(core_axis_name="c", subcore_axis_name="s")
    run = pl.kernel(
        _body,
        out_type=jax.ShapeDtypeStruct((B, D), jnp.float32),
        mesh=mesh,
        scratch_types=[
            pltpu.VMEM((rows_per_w // _IPG, _IPG), jnp.int32),
            pltpu.VMEM((_RING, _CHUNK, D), jnp.float32),
            pltpu.SemaphoreType.DMA((_RING,)),
            pltpu.SemaphoreType.DMA((_RING,)),
        ],
        compiler_params=pltpu.CompilerParams(use_tc_tiling_on_sc=False),
    )
    out = run(idx, w_rows)
    return out.reshape(S, T, D)


# speedup vs baseline: 1.1001x; 1.1001x over previous
import jax
import jax.numpy as jnp
from jax import lax
from jax.experimental import pallas as pl
from jax.experimental.pallas import tpu as pltpu
from jax.experimental.pallas import tpu_sc as plsc

_NC = 2
_NS = 16
_NW = _NC * _NS

_CHUNK = 256
_IPG = 128
_GPC = _CHUNK // _IPG
_RING = 4

_TPACK = 500


def _pack_body(x_ref, y_ref):
    x = x_ref[...]
    for i in range(8):
        y_ref[:, i, 0:64] = x[:, (2 * i) // 8, (2 * i) % 8, :]
        y_ref[:, i, 64:128] = x[:, (2 * i + 1) // 8, (2 * i + 1) % 8, :]


def _pack(w4):
    G = w4.shape[0]
    return pl.pallas_call(
        _pack_body,
        grid=(G // _TPACK,),
        in_specs=[pl.BlockSpec((_TPACK, 2, 8, 64), lambda j: (j, 0, 0, 0))],
        out_specs=pl.BlockSpec((_TPACK, 8, 128), lambda j: (j, 0, 0)),
        out_shape=jax.ShapeDtypeStruct((G, 8, 128), jnp.float32),
        compiler_params=pltpu.CompilerParams(
            vmem_limit_bytes=100 * 1024 * 1024
        ),
    )(w4)


def _body(idx_hbm, table_hbm, out_hbm, idx_v, rows_v, gsem, osem):
    cpw = idx_hbm.shape[1] // _GPC
    wid = lax.axis_index("s") * _NC + lax.axis_index("c")
    pltpu.sync_copy(idx_hbm.at[wid], idx_v)
    out_base = wid * cpw

    def fire_gathers(g, b):
        for j in range(_GPC):
            pltpu.async_copy(
                table_hbm.at[idx_v.at[g * _GPC + j]],
                rows_v.at[b, pl.ds(j * _IPG, _IPG)],
                gsem.at[b],
            )

    def drain_gathers(b):
        pltpu.make_async_copy(
            out_hbm.at[pl.ds(0, _CHUNK)], rows_v.at[b], gsem.at[b]
        ).wait()

    def out_slice(g):
        base = pl.multiple_of((out_base + g) * _CHUNK, _CHUNK)
        return out_hbm.at[pl.ds(base, _CHUNK)]

    for b in range(_RING):
        fire_gathers(b, b)

    @pl.loop(0, cpw - _RING, step=_RING)
    def _steady(i):
        for b in range(_RING):
            g = i + b
            drain_gathers(b)
            pltpu.async_copy(rows_v.at[b], out_slice(g), osem.at[b])
            pltpu.make_async_copy(
                rows_v.at[b], out_hbm.at[pl.ds(0, _CHUNK)], osem.at[b]
            ).wait()
            fire_gathers(g + _RING, b)

    for b in range(_RING):
        g = cpw - _RING + b
        drain_gathers(b)
        pltpu.sync_copy(rows_v.at[b], out_slice(g))


def kernel(token_ids, W):
    S, T = token_ids.shape
    D = W.shape[1]
    B = S * T
    rows_per_w = B // _NW
    idx = token_ids.astype(jnp.int32).reshape(_NW, rows_per_w // _IPG, _IPG)

    w4 = W.reshape(W.shape[0] // 16, 2, 8, D)
    w_rows = _pack(w4).reshape(W.shape[0], D)

    mesh = plsc.VectorSubcoreMesh(core_axis_name="c", subcore_axis_name="s")
    run = pl.kernel(
        _body,
        out_type=jax.ShapeDtypeStruct((B, D), jnp.float32),
        mesh=mesh,
        scratch_types=[
            pltpu.VMEM((rows_per_w // _IPG, _IPG), jnp.int32),
            pltpu.VMEM((_RING, _CHUNK, D), jnp.float32),
            pltpu.SemaphoreType.DMA((_RING,)),
            pltpu.SemaphoreType.DMA((_RING,)),
        ],
        compiler_params=pltpu.CompilerParams(use_tc_tiling_on_sc=False),
    )
    out = run(idx, w_rows)
    return out.reshape(S, T, D)


# speedup vs baseline: 1.1203x; 1.0183x over previous
import jax
import jax.numpy as jnp
from jax import lax
from jax.experimental import pallas as pl
from jax.experimental.pallas import tpu as pltpu
from jax.experimental.pallas import tpu_sc as plsc

_NC = 2
_NS = 16
_NW = _NC * _NS

_CHUNK = 256
_IPG = 128
_GPC = _CHUNK // _IPG
_RING = 4

_TPACK = 625


def _pack_body(x_ref, y_ref):
    x = x_ref[...]
    for i in range(8):
        y_ref[:, i, 0:64] = x[:, (2 * i) // 8, (2 * i) % 8, :]
        y_ref[:, i, 64:128] = x[:, (2 * i + 1) // 8, (2 * i + 1) % 8, :]


def _pack(w4):
    G = w4.shape[0]
    return pl.pallas_call(
        _pack_body,
        grid=(G // _TPACK,),
        in_specs=[pl.BlockSpec((_TPACK, 2, 8, 64), lambda j: (j, 0, 0, 0))],
        out_specs=pl.BlockSpec((_TPACK, 8, 128), lambda j: (j, 0, 0)),
        out_shape=jax.ShapeDtypeStruct((G, 8, 128), jnp.float32),
        compiler_params=pltpu.CompilerParams(
            vmem_limit_bytes=100 * 1024 * 1024
        ),
    )(w4)


def _body(idx_hbm, table_hbm, out_hbm, idx_v, rows_v, gsem, osem):
    cpw = idx_hbm.shape[1] // _GPC
    wid = lax.axis_index("s") * _NC + lax.axis_index("c")
    pltpu.sync_copy(idx_hbm.at[wid], idx_v)
    out_base = wid * cpw

    def fire_gathers(g, b):
        for j in range(_GPC):
            pltpu.async_copy(
                table_hbm.at[idx_v.at[g * _GPC + j]],
                rows_v.at[b, pl.ds(j * _IPG, _IPG)],
                gsem.at[b],
            )

    def drain_gathers(b):
        pltpu.make_async_copy(
            out_hbm.at[pl.ds(0, _CHUNK)], rows_v.at[b], gsem.at[b]
        ).wait()

    def out_slice(g):
        base = pl.multiple_of((out_base + g) * _CHUNK, _CHUNK)
        return out_hbm.at[pl.ds(base, _CHUNK)]

    for b in range(_RING):
        fire_gathers(b, b)

    @pl.loop(0, cpw - _RING, step=_RING)
    def _steady(i):
        for b in range(_RING):
            g = i + b
            drain_gathers(b)
            pltpu.async_copy(rows_v.at[b], out_slice(g), osem.at[b])
            pltpu.make_async_copy(
                rows_v.at[b], out_hbm.at[pl.ds(0, _CHUNK)], osem.at[b]
            ).wait()
            fire_gathers(g + _RING, b)

    for b in range(_RING):
        g = cpw - _RING + b
        drain_gathers(b)
        pltpu.sync_copy(rows_v.at[b], out_slice(g))


def kernel(token_ids, W):
    S, T = token_ids.shape
    D = W.shape[1]
    B = S * T
    rows_per_w = B // _NW
    idx = token_ids.astype(jnp.int32).reshape(_NW, rows_per_w // _IPG, _IPG)

    w4 = W.reshape(W.shape[0] // 16, 2, 8, D)
    w_rows = _pack(w4).reshape(W.shape[0], D)

    mesh = plsc.VectorSubcoreMesh(core_axis_name="c", subcore_axis_name="s")
    run = pl.kernel(
        _body,
        out_type=jax.ShapeDtypeStruct((B, D), jnp.float32),
        mesh=mesh,
        scratch_types=[
            pltpu.VMEM((rows_per_w // _IPG, _IPG), jnp.int32),
            pltpu.VMEM((_RING, _CHUNK, D), jnp.float32),
            pltpu.SemaphoreType.DMA((_RING,)),
            pltpu.SemaphoreType.DMA((_RING,)),
        ],
        compiler_params=pltpu.CompilerParams(use_tc_tiling_on_sc=False),
    )
    out = run(idx, w_rows)
    return out.reshape(S, T, D)


# speedup vs baseline: 1.1548x; 1.0308x over previous
import jax
import jax.numpy as jnp
from jax import lax
from jax.experimental import pallas as pl
from jax.experimental.pallas import tpu as pltpu
from jax.experimental.pallas import tpu_sc as plsc

_NC = 2
_NS = 16
_NW = _NC * _NS

_CHUNK = 256
_IPG = 128
_GPC = _CHUNK // _IPG
_RING = 4

_TPACK = 1250


def _pack_body(x_ref, y_ref):
    x = x_ref[...]
    for i in range(8):
        y_ref[:, i, 0:64] = x[:, (2 * i) // 8, (2 * i) % 8, :]
        y_ref[:, i, 64:128] = x[:, (2 * i + 1) // 8, (2 * i + 1) % 8, :]


def _pack(w4):
    G = w4.shape[0]
    return pl.pallas_call(
        _pack_body,
        grid=(G // _TPACK,),
        in_specs=[pl.BlockSpec((_TPACK, 2, 8, 64), lambda j: (j, 0, 0, 0))],
        out_specs=pl.BlockSpec((_TPACK, 8, 128), lambda j: (j, 0, 0)),
        out_shape=jax.ShapeDtypeStruct((G, 8, 128), jnp.float32),
        compiler_params=pltpu.CompilerParams(
            vmem_limit_bytes=100 * 1024 * 1024
        ),
    )(w4)


def _body(idx_hbm, table_hbm, out_hbm, idx_v, rows_v, gsem, osem):
    cpw = idx_hbm.shape[1] // _GPC
    wid = lax.axis_index("s") * _NC + lax.axis_index("c")
    pltpu.sync_copy(idx_hbm.at[wid], idx_v)
    out_base = wid * cpw

    def fire_gathers(g, b):
        for j in range(_GPC):
            pltpu.async_copy(
                table_hbm.at[idx_v.at[g * _GPC + j]],
                rows_v.at[b, pl.ds(j * _IPG, _IPG)],
                gsem.at[b],
            )

    def drain_gathers(b):
        pltpu.make_async_copy(
            out_hbm.at[pl.ds(0, _CHUNK)], rows_v.at[b], gsem.at[b]
        ).wait()

    def out_slice(g):
        base = pl.multiple_of((out_base + g) * _CHUNK, _CHUNK)
        return out_hbm.at[pl.ds(base, _CHUNK)]

    for b in range(_RING):
        fire_gathers(b, b)

    @pl.loop(0, cpw - _RING, step=_RING)
    def _steady(i):
        for b in range(_RING):
            g = i + b
            drain_gathers(b)
            pltpu.async_copy(rows_v.at[b], out_slice(g), osem.at[b])
            pltpu.make_async_copy(
                rows_v.at[b], out_hbm.at[pl.ds(0, _CHUNK)], osem.at[b]
            ).wait()
            fire_gathers(g + _RING, b)

    for b in range(_RING):
        g = cpw - _RING + b
        drain_gathers(b)
        pltpu.sync_copy(rows_v.at[b], out_slice(g))


def kernel(token_ids, W):
    S, T = token_ids.shape
    D = W.shape[1]
    B = S * T
    rows_per_w = B // _NW
    idx = token_ids.astype(jnp.int32).reshape(_NW, rows_per_w // _IPG, _IPG)

    w4 = W.reshape(W.shape[0] // 16, 2, 8, D)
    w_rows = _pack(w4).reshape(W.shape[0], D)

    mesh = plsc.VectorSubcoreMesh(core_axis_name="c", subcore_axis_name="s")
    run = pl.kernel(
        _body,
        out_type=jax.ShapeDtypeStruct((B, D), jnp.float32),
        mesh=mesh,
        scratch_types=[
            pltpu.VMEM((rows_per_w // _IPG, _IPG), jnp.int32),
            pltpu.VMEM((_RING, _CHUNK, D), jnp.float32),
            pltpu.SemaphoreType.DMA((_RING,)),
            pltpu.SemaphoreType.DMA((_RING,)),
        ],
        compiler_params=pltpu.CompilerParams(use_tc_tiling_on_sc=False),
    )
    out = run(idx, w_rows)
    return out.reshape(S, T, D)
